# baseline (speedup 1.0000x reference)
import functools

import jax
import jax.numpy as jnp
from jax import lax
from jax.experimental import pallas as pl
from jax.experimental.pallas import tpu as pltpu
from jax.experimental.pallas import tpu_sc as plsc

_D = 16
_R = 8
_SD = _D * _R
_NC = 2
_NS = 16
_CHUNK = 512
_LCH = 512
_TAIL = 64
_SPLIT = 499712

_params = pltpu.CompilerParams(
    use_tc_tiling_on_sc=True, needs_layout_passes=False
)


def _sc_relayout_gather(wT, idx_flat):
    n = wT.shape[1]
    n_idx = idx_flat.shape[0]
    b_per_w = n_idx // _NS
    n_gchunks = b_per_w // _CHUNK
    half_chunks = (_SPLIT // _LCH, (n - _TAIL - _SPLIT) // _LCH)
    half_supers = (_SPLIT // _R, (n - _TAIL - _SPLIT) // _R)
    max_chunks = max(half_chunks)
    mesh = plsc.VectorSubcoreMesh(core_axis_name="c", subcore_axis_name="s")

    @functools.partial(
        pl.kernel,
        mesh=mesh,
        out_type=(
            jax.ShapeDtypeStruct((_NC, _D, n_idx), jnp.float32),
            jax.ShapeDtypeStruct((n // _R, _SD), jnp.float32),
        ),
        compiler_params=_params,
        scratch_types=[
            pltpu.VMEM((_D, _LCH), jnp.float32),
            pltpu.VMEM((_LCH // _R, _SD), jnp.float32),
            pltpu.VMEM((b_per_w,), jnp.int32),
            pltpu.VMEM((b_per_w,), jnp.int32),
            pltpu.VMEM((_CHUNK, _SD), jnp.float32),
            pltpu.VMEM((_D, b_per_w), jnp.float32),
            pltpu.SemaphoreType.DMA,
        ],
    )
    def k(wT_hbm, idx_hbm, out_hbm, w8_hbm,
          in_v, tr_v, idx_v, sidx_v, rows_v, comp_v, sem):
        core = lax.axis_index("c")
        sub = lax.axis_index("s")
        n_my = jnp.where(core == 0, half_chunks[0], half_chunks[1])
        c0 = jnp.where(core == 0, 0, half_chunks[0])
        rows16 = lax.iota(jnp.int32, _D)

        @pl.loop(sub, max_chunks, step=_NS)
        def _(ci):
            @pl.when(ci < n_my)
            def _():
                c = c0 + ci
                pltpu.sync_copy(wT_hbm.at[:, pl.ds(c * _LCH, _LCH)], in_v)

                @functools.partial(
                    plsc.parallel_loop, 0, _LCH // _R, unroll=8
                )
                def _(s):
                    base = s * _R
                    for a in range(_R):
                        col = jnp.full((_D,), base + a, jnp.int32)
                        tr_v[s, pl.ds(a * _D, _D)] = plsc.load_gather(
                            in_v, [rows16, col]
                        )

                pltpu.sync_copy(
                    tr_v, w8_hbm.at[pl.ds(c * (_LCH // _R), _LCH // _R)]
                )

        plsc.subcore_barrier()

        base = sub * b_per_w
        pltpu.sync_copy(idx_hbm.at[pl.ds(base, b_per_w)], idx_v)
        s_lo = c0 * (_LCH // _R)
        s_num = jnp.where(core == 0, half_supers[0], half_supers[1])

        @plsc.parallel_loop(0, b_per_w, _D, unroll=8)
        def _(j):
            s = lax.shift_right_logical(idx_v[pl.ds(j, _D)], 3) - s_lo
            sidx_v[pl.ds(j, _D)] = jnp.clip(s, 0, s_num - 1) + s_lo

        lane = lax.iota(jnp.int32, _D)

        @pl.loop(0, n_gchunks)
        def _(c):
            pltpu.async_copy(
                w8_hbm.at[sidx_v.at[pl.ds(c * _CHUNK, _CHUNK)]], rows_v, sem
            ).wait()

            @pl.loop(0, _CHUNK, step=_D)
            def _(j0):
                g = c * _CHUNK + j0
                cols = (idx_v[pl.ds(g, _D)] & 7) * _D
                r16 = lane + j0
                for kk in range(_D):
                    comp_v[kk, pl.ds(g, _D)] = plsc.load_gather(
                        rows_v, [r16, cols + kk]
                    )

        pltpu.sync_copy(comp_v, out_hbm.at[core, :, pl.ds(base, b_per_w)])

    return k(wT, idx_flat)


def _hdist_body(t0, u0_ref, v0_ref, u1_ref, v1_ref, i_ref, t_ref, s_ref,
                o_ref):
    blk = u0_ref.shape[2]
    ii = i_ref[...]
    wt = t_ref[...]

    def pick(x0, x1, col):
        x = jnp.where((col < _SPLIT)[None, :], x0, x1)
        m = col >= t0
        loc = jnp.where(m, col - t0, 0)
        loc2 = jnp.broadcast_to(loc[None, :], (_D, blk))
        fx = jnp.take_along_axis(wt, loc2, axis=1)
        return jnp.where(m[None, :], fx, x)

    u = pick(u0_ref[0], u1_ref[0], ii[:, 0])
    v = pick(v0_ref[0], v1_ref[0], ii[:, 1])
    su = jnp.sum(u * u, axis=0)
    sv = jnp.sum(v * v, axis=0)
    d = u - v
    z = 2.0 * jnp.sum(d * d, axis=0)
    uu = 1.0 + z / ((1.0 - su) * (1.0 - sv))
    acosh = jnp.log(uu + jnp.sqrt(uu * uu - 1.0))
    o_ref[...] = acosh / (1.0 + s_ref[0])


def _tc_math(g, idx, wtail, scale, b, blk, t0):
    nb = b // blk
    return pl.pallas_call(
        functools.partial(_hdist_body, t0),
        grid=(nb,),
        in_specs=[
            pl.BlockSpec((1, _D, blk), lambda i: (0, 0, i)),
            pl.BlockSpec((1, _D, blk), lambda i: (0, 0, i + nb)),
            pl.BlockSpec((1, _D, blk), lambda i: (1, 0, i)),
            pl.BlockSpec((1, _D, blk), lambda i: (1, 0, i + nb)),
            pl.BlockSpec((blk, 2), lambda i: (i, 0)),
            pl.BlockSpec((_D, _TAIL), lambda i: (0, 0)),
            pl.BlockSpec(memory_space=pltpu.SMEM),
        ],
        out_specs=pl.BlockSpec((blk,), lambda i: (i,)),
        out_shape=jax.ShapeDtypeStruct((b,), jnp.float32),
    )(g, g, g, g, idx, wtail, scale)


def kernel(idx, w, scale):
    b = idx.shape[0]
    n = w.shape[0]
    t0 = n - _TAIL
    idx = idx.astype(jnp.int32)
    idx_flat = jnp.concatenate([idx[:, 0], idx[:, 1]])
    wT = w.T
    wtail = wT[:, t0:]
    g, _unused_w8 = _sc_relayout_gather(wT, idx_flat)
    return _tc_math(g, idx, wtail, scale, b, blk=512, t0=t0)


# speedup vs baseline: 2.3633x; 2.3633x over previous
import functools

import jax
import jax.numpy as jnp
from jax import lax
from jax.experimental import pallas as pl
from jax.experimental.pallas import tpu as pltpu
from jax.experimental.pallas import tpu_sc as plsc

_D = 16
_R = 8
_SD = _D * _R
_NC = 2
_NS = 16
_NW = _NC * _NS
_CHUNK = 512


def _sc_gather(w8, idx_flat):
    n_idx = idx_flat.shape[0]
    b_per_w = n_idx // _NW
    n_chunks = b_per_w // _CHUNK
    mesh = plsc.VectorSubcoreMesh(core_axis_name="c", subcore_axis_name="s")

    @functools.partial(
        pl.kernel,
        mesh=mesh,
        out_type=jax.ShapeDtypeStruct((_D, n_idx), jnp.float32),
        compiler_params=pltpu.CompilerParams(
            use_tc_tiling_on_sc=True, needs_layout_passes=False
        ),
        scratch_types=[
            pltpu.VMEM((b_per_w,), jnp.int32),
            pltpu.VMEM((b_per_w,), jnp.int32),
            pltpu.VMEM((_CHUNK, _SD), jnp.float32),
            pltpu.VMEM((_D, b_per_w), jnp.float32),
            pltpu.SemaphoreType.DMA,
        ],
    )
    def gather_k(w_hbm, idx_hbm, out_hbm, idx_v, sidx_v, rows_v, comp_v, sem):
        wid = lax.axis_index("s") * _NC + lax.axis_index("c")
        base = wid * b_per_w
        pltpu.sync_copy(idx_hbm.at[pl.ds(base, b_per_w)], idx_v)

        @pl.loop(0, b_per_w, step=_D)
        def _(j):
            sidx_v[pl.ds(j, _D)] = lax.shift_right_logical(
                idx_v[pl.ds(j, _D)], 3
            )

        lane = lax.iota(jnp.int32, _D)

        @pl.loop(0, n_chunks)
        def _(c):
            pltpu.async_copy(
                w_hbm.at[sidx_v.at[pl.ds(c * _CHUNK, _CHUNK)]], rows_v, sem
            ).wait()

            @pl.loop(0, _CHUNK, step=_D)
            def _(j0):
                g = c * _CHUNK + j0
                cols = (idx_v[pl.ds(g, _D)] & 7) * _D
                rows16 = lane + j0
                for k in range(_D):
                    comp_v[k, pl.ds(g, _D)] = plsc.load_gather(
                        rows_v, [rows16, cols + k]
                    )

        pltpu.sync_copy(comp_v, out_hbm.at[:, pl.ds(base, b_per_w)])

    return gather_k(w8, idx_flat)


def _hdist_body(u_ref, v_ref, s_ref, o_ref):
    u = u_ref[...]
    v = v_ref[...]
    su = jnp.sum(u * u, axis=0)
    sv = jnp.sum(v * v, axis=0)
    d = u - v
    z = 2.0 * jnp.sum(d * d, axis=0)
    uu = 1.0 + z / ((1.0 - su) * (1.0 - sv))
    acosh = jnp.log(uu + jnp.sqrt(uu * uu - 1.0))
    o_ref[...] = acosh / (1.0 + s_ref[0])


def _tc_math(g, scale, b, blk):
    nb = b // blk
    return pl.pallas_call(
        _hdist_body,
        grid=(nb,),
        in_specs=[
            pl.BlockSpec((_D, blk), lambda i: (0, i)),
            pl.BlockSpec((_D, blk), lambda i: (0, i + nb)),
            pl.BlockSpec(memory_space=pltpu.SMEM),
        ],
        out_specs=pl.BlockSpec((blk,), lambda i: (i,)),
        out_shape=jax.ShapeDtypeStruct((b,), jnp.float32),
    )(g, g, scale)


def kernel(idx, w, scale):
    b = idx.shape[0]
    idx = idx.astype(jnp.int32)
    idx_flat = jnp.concatenate([idx[:, 0], idx[:, 1]])
    w8 = w.reshape(w.shape[0] // _R, _SD)
    g = _sc_gather(w8, idx_flat)
    return _tc_math(g, scale, b, blk=2048)
